# baseline (speedup 1.0000x reference)
import jax
import jax.numpy as jnp
import numpy as np
from jax.experimental import pallas as pl
from jax.experimental.pallas import tpu as pltpu

N_GENES = 10000
N_RXN = 10000
N_MET = 10000
C = 128
NH = 2
DH = C // NH
L = 128
BS = 8

_pallas_call = pl.pallas_call


def _ln(x, w, b):
    mu = jnp.mean(x, axis=-1, keepdims=True)
    var = jnp.mean((x - mu) * (x - mu), axis=-1, keepdims=True)
    return (x - mu) * jax.lax.rsqrt(var + 1e-5) * w + b


def _mha_blk(x, y, mask_y, in_w, in_b, out_w, out_b):
    B, Lq, _ = x.shape
    Lk = y.shape[1]
    x2 = x.reshape(B * Lq, C)
    y2 = y.reshape(B * Lk, C)
    q = (x2 @ in_w[:C].T + in_b[:C]).reshape(B, Lq, C)
    k = (y2 @ in_w[C:2 * C].T + in_b[C:2 * C]).reshape(B, Lk, C)
    v = (y2 @ in_w[2 * C:].T + in_b[2 * C:]).reshape(B, Lk, C)
    outs = []
    scale = 1.0 / np.sqrt(DH)
    for h in range(NH):
        qh = q[:, :, h * DH:(h + 1) * DH]
        kh = k[:, :, h * DH:(h + 1) * DH]
        vh = v[:, :, h * DH:(h + 1) * DH]
        s = jax.lax.dot_general(
            qh, kh, (((2,), (2,)), ((0,), (0,))),
            preferred_element_type=jnp.float32) * scale
        if mask_y is not None:
            s = jnp.where(mask_y[:, None, :] > 0, s, -1e9)
        s = s - jnp.max(s, axis=-1, keepdims=True)
        e = jnp.exp(s)
        a = e / jnp.sum(e, axis=-1, keepdims=True)
        oh = jax.lax.dot_general(
            a, vh, (((2,), (1,)), ((0,), (0,))),
            preferred_element_type=jnp.float32)
        outs.append(oh)
    o = jnp.concatenate(outs, axis=-1).reshape(B * Lq, C)
    return (o @ out_w.T + out_b).reshape(B, Lq, C)


def _mab_blk(x, y, mask_x, mask_y, P, i):
    out = _mha_blk(x, y, mask_y, P['in_w'][i], P['in_b'][i],
                   P['out_w'][i], P['out_b'][i])
    if mask_x is not None:
        out = out * mask_x[:, :, None]
    out = out + x
    out = _ln(out, P['ln1_w'][i], P['ln1_b'][i])
    B, Lq, _ = out.shape
    o2 = out.reshape(B * Lq, C)
    out = out + jax.nn.relu(o2 @ P['lin_w'][i].T + P['lin_b'][i]).reshape(B, Lq, C)
    out = _ln(out, P['ln2_w'][i], P['ln2_b'][i])
    return out


def _st_kernel(x_ref, mask_ref, in_w, in_b, out_w, out_b, lin_w, lin_b,
               l1w, l1b, l2w, l2b, pma_w, pma_b, seed, o_ref):
    B = x_ref.shape[0]
    x = x_ref[...]
    mask = mask_ref[...]
    P = {'in_w': in_w, 'in_b': in_b, 'out_w': out_w, 'out_b': out_b,
         'lin_w': lin_w, 'lin_b': lin_b, 'ln1_w': l1w, 'ln1_b': l1b,
         'ln2_w': l2w, 'ln2_b': l2b}
    for i in (0, 1):
        x = _mab_blk(x, x, mask, mask, P, i)
    xx = jax.nn.relu(x.reshape(B * L, C) @ pma_w[...].T + pma_b[0]).reshape(B, L, C)
    sd = jnp.broadcast_to(seed[...].reshape(1, 1, C), (B, 1, C))
    x = _mab_blk(sd, xx, None, mask, P, 2)
    x = _mab_blk(x, x, None, None, P, 3)
    x = jnp.nan_to_num(x)
    o_ref[...] = x.reshape(B, C)


def _stack_st_params(p):
    mabs = [p['encoders'][0], p['encoders'][1], p['pma_mab'], p['decoders'][0]]
    stk = lambda f: jnp.stack([m[f] for m in mabs])
    return (stk('in_w'), stk('in_b'), stk('out_w'), stk('out_b'),
            stk('lin_w'), stk('lin_b'), stk('ln1_w'), stk('ln1_b'),
            stk('ln2_w'), stk('ln2_b'),
            p['pma_lin_w'], p['pma_lin_b'].reshape(1, C),
            p['seed'].reshape(1, C))


def _st_apply(x_flat, index, counts, S, p):
    starts = jnp.cumsum(counts) - counts
    pos = jnp.arange(index.shape[0], dtype=index.dtype) - starts[index].astype(index.dtype)
    dense = jnp.zeros((S, L, C), x_flat.dtype).at[index, pos].set(x_flat)
    mask = (jnp.arange(L)[None, :] < counts[:, None]).astype(jnp.float32)
    prm = _stack_st_params(p)
    full = lambda a: pl.BlockSpec(a.shape, lambda i: (0,) * a.ndim)
    out = _pallas_call(
        _st_kernel,
        grid=(S // BS,),
        in_specs=[
            pl.BlockSpec((BS, L, C), lambda i: (i, 0, 0)),
            pl.BlockSpec((BS, L), lambda i: (i, 0)),
        ] + [full(a) for a in prm],
        out_specs=pl.BlockSpec((BS, C), lambda i: (i, 0)),
        out_shape=jax.ShapeDtypeStruct((S, C), jnp.float32),
        compiler_params=pltpu.CompilerParams(
            dimension_semantics=("arbitrary",)),
    )(dense, mask, *prm)
    return out


def _segment_softmax(alpha, index, num_segments):
    amax = jax.ops.segment_max(alpha, index, num_segments)
    amax = jnp.where(jnp.isfinite(amax), amax, 0.0)
    e = jnp.exp(alpha - amax[index])
    denom = jax.ops.segment_sum(e, index, num_segments) + 1e-16
    return e / denom[index]


def _conv(x, src, dst, stoich, he_attr, p):
    n_nodes = x.shape[0]
    n_edges = he_attr.shape[0]
    xt = x @ p['lin_w'].T
    he = he_attr @ p['lin_w'].T
    a_src = xt @ p['att'][:C]
    a_dst = he @ p['att'][C:]
    alpha = jax.nn.leaky_relu(a_src[src] + a_dst[dst], 0.2)
    alpha = _segment_softmax(alpha, dst, n_edges)
    ast = jnp.abs(stoich)
    Dd = jax.ops.segment_sum(ast, src, n_nodes)
    Dinv = jnp.where(Dd > 0, 1.0 / Dd, 0.0)
    Bb = jax.ops.segment_sum(ast, dst, n_edges)
    Binv = jnp.where(Bb > 0, 1.0 / Bb, 0.0)
    w1 = stoich * Binv[dst] * alpha
    edge_feat = jax.ops.segment_sum(w1[:, None] * xt[src], dst, n_edges)
    w2 = stoich * Dinv[src] * alpha
    out = jax.ops.segment_sum(w2[:, None] * edge_feat[dst], src, n_nodes)
    return out + p['bias']


def kernel(gene_x, stoich, params, gpr_edge_index, met_edge_index):
    perm = jnp.argsort(gpr_edge_index[1])
    g_idx = gpr_edge_index[0][perm]
    r_idx = gpr_edge_index[1][perm]
    cnt_r = jnp.bincount(r_idx, length=N_RXN)
    H_r = _st_apply(gene_x[g_idx], r_idx, cnt_r, N_RXN, params['gr_st'])

    emb = params['emb']
    norms = jnp.linalg.norm(emb, axis=1, keepdims=True)
    Z_m = emb * jnp.where(norms > 1.0, 1.0 / (norms + 1e-7), 1.0)

    perm2 = jnp.argsort(met_edge_index[1])
    m_idx = met_edge_index[0][perm2]
    mr_idx = met_edge_index[1][perm2]
    st = stoich[perm2]
    for cp in params['convs']:
        Z_m = _conv(Z_m, m_idx, mr_idx, st, H_r, cp)

    cnt_mr = jnp.bincount(mr_idx, length=N_RXN)
    Z_r = _st_apply(Z_m[m_idx], mr_idx, cnt_mr, N_RXN, params['rm_st'])

    perm3 = jnp.argsort(g_idx)
    gg = g_idx[perm3]
    rr = r_idx[perm3]
    cnt_g = jnp.bincount(gg, length=N_GENES)
    Z_mg = _st_apply(Z_r[rr], gg, cnt_g, N_GENES, params['gr_st'])
    return Z_mg


# speedup vs baseline: 1.1362x; 1.1362x over previous
import jax
import jax.numpy as jnp
import numpy as np
from jax.experimental import pallas as pl
from jax.experimental.pallas import tpu as pltpu

N_GENES = 10000
N_RXN = 10000
N_MET = 10000
C = 128
NH = 2
DH = C // NH
L = 128
BS = 16

_pallas_call = pl.pallas_call


def _ln(x, w, b):
    mu = jnp.mean(x, axis=-1, keepdims=True)
    var = jnp.mean((x - mu) * (x - mu), axis=-1, keepdims=True)
    return (x - mu) * jax.lax.rsqrt(var + 1e-5) * w + b


def _mha_blk(x, y, mask_y, in_w, in_b, out_w, out_b):
    B, Lq, _ = x.shape
    Lk = y.shape[1]
    x2 = x.reshape(B * Lq, C)
    y2 = y.reshape(B * Lk, C)
    q = (x2 @ in_w[:C].T + in_b[:C]).reshape(B, Lq, C)
    k = (y2 @ in_w[C:2 * C].T + in_b[C:2 * C]).reshape(B, Lk, C)
    v = (y2 @ in_w[2 * C:].T + in_b[2 * C:]).reshape(B, Lk, C)
    acc = jnp.broadcast_to(out_b, (B * Lq, C))
    scale = 1.0 / np.sqrt(DH)
    for h in range(NH):
        qh = q[:, :, h * DH:(h + 1) * DH]
        kh = k[:, :, h * DH:(h + 1) * DH]
        vh = v[:, :, h * DH:(h + 1) * DH]
        s = jax.lax.dot_general(
            qh, kh, (((2,), (2,)), ((0,), (0,))),
            preferred_element_type=jnp.float32) * scale
        if mask_y is not None:
            s = jnp.where(mask_y[:, None, :] > 0, s, -1e9)
        s = s - jnp.max(s, axis=-1, keepdims=True)
        e = jnp.exp(s)
        a = e / jnp.sum(e, axis=-1, keepdims=True)
        oh = jax.lax.dot_general(
            a, vh, (((2,), (1,)), ((0,), (0,))),
            preferred_element_type=jnp.float32)
        acc = acc + oh.reshape(B * Lq, DH) @ out_w[:, h * DH:(h + 1) * DH].T
    return acc.reshape(B, Lq, C)


def _mab_blk(x, y, mask_x, mask_y, P, i):
    out = _mha_blk(x, y, mask_y, P['in_w'][i], P['in_b'][i],
                   P['out_w'][i], P['out_b'][i])
    if mask_x is not None:
        out = out * mask_x[:, :, None]
    out = out + x
    out = _ln(out, P['ln1_w'][i], P['ln1_b'][i])
    B, Lq, _ = out.shape
    o2 = out.reshape(B * Lq, C)
    out = out + jax.nn.relu(o2 @ P['lin_w'][i].T + P['lin_b'][i]).reshape(B, Lq, C)
    out = _ln(out, P['ln2_w'][i], P['ln2_b'][i])
    return out


def _st_kernel(x_ref, mask_ref, in_w, in_b, out_w, out_b, lin_w, lin_b,
               l1w, l1b, l2w, l2b, pma_w, pma_b, seed, o_ref):
    B = x_ref.shape[0]
    x = x_ref[...]
    mask = mask_ref[...]
    P = {'in_w': in_w, 'in_b': in_b, 'out_w': out_w, 'out_b': out_b,
         'lin_w': lin_w, 'lin_b': lin_b, 'ln1_w': l1w, 'ln1_b': l1b,
         'ln2_w': l2w, 'ln2_b': l2b}
    for i in (0, 1):
        x = _mab_blk(x, x, mask, mask, P, i)
    xx = jax.nn.relu(x.reshape(B * L, C) @ pma_w[...].T + pma_b[0]).reshape(B, L, C)
    sd = jnp.broadcast_to(seed[...].reshape(1, 1, C), (B, 1, C))
    x = _mab_blk(sd, xx, None, mask, P, 2)
    x = _mab_blk(x, x, None, None, P, 3)
    x = jnp.nan_to_num(x)
    o_ref[...] = x.reshape(B, C)


def _stack_st_params(p):
    mabs = [p['encoders'][0], p['encoders'][1], p['pma_mab'], p['decoders'][0]]
    stk = lambda f: jnp.stack([m[f] for m in mabs])
    return (stk('in_w'), stk('in_b'), stk('out_w'), stk('out_b'),
            stk('lin_w'), stk('lin_b'), stk('ln1_w'), stk('ln1_b'),
            stk('ln2_w'), stk('ln2_b'),
            p['pma_lin_w'], p['pma_lin_b'].reshape(1, C),
            p['seed'].reshape(1, C))


def _st_apply(x_flat, index, counts, S, p):
    starts = jnp.cumsum(counts) - counts
    pos = jnp.arange(index.shape[0], dtype=index.dtype) - starts[index].astype(index.dtype)
    dense = jnp.zeros((S, L, C), x_flat.dtype).at[index, pos].set(x_flat)
    mask = (jnp.arange(L)[None, :] < counts[:, None]).astype(jnp.float32)
    prm = _stack_st_params(p)
    full = lambda a: pl.BlockSpec(a.shape, lambda i: (0,) * a.ndim)
    out = _pallas_call(
        _st_kernel,
        grid=(S // BS,),
        in_specs=[
            pl.BlockSpec((BS, L, C), lambda i: (i, 0, 0)),
            pl.BlockSpec((BS, L), lambda i: (i, 0)),
        ] + [full(a) for a in prm],
        out_specs=pl.BlockSpec((BS, C), lambda i: (i, 0)),
        out_shape=jax.ShapeDtypeStruct((S, C), jnp.float32),
        compiler_params=pltpu.CompilerParams(
            dimension_semantics=("parallel",)),
    )(dense, mask, *prm)
    return out


def _segment_softmax(alpha, index, num_segments):
    amax = jax.ops.segment_max(alpha, index, num_segments)
    amax = jnp.where(jnp.isfinite(amax), amax, 0.0)
    e = jnp.exp(alpha - amax[index])
    denom = jax.ops.segment_sum(e, index, num_segments) + 1e-16
    return e / denom[index]


def _conv(x, src, dst, stoich, he_attr, p):
    n_nodes = x.shape[0]
    n_edges = he_attr.shape[0]
    xt = x @ p['lin_w'].T
    he = he_attr @ p['lin_w'].T
    a_src = xt @ p['att'][:C]
    a_dst = he @ p['att'][C:]
    alpha = jax.nn.leaky_relu(a_src[src] + a_dst[dst], 0.2)
    alpha = _segment_softmax(alpha, dst, n_edges)
    ast = jnp.abs(stoich)
    Dd = jax.ops.segment_sum(ast, src, n_nodes)
    Dinv = jnp.where(Dd > 0, 1.0 / Dd, 0.0)
    Bb = jax.ops.segment_sum(ast, dst, n_edges)
    Binv = jnp.where(Bb > 0, 1.0 / Bb, 0.0)
    w1 = stoich * Binv[dst] * alpha
    edge_feat = jax.ops.segment_sum(w1[:, None] * xt[src], dst, n_edges)
    w2 = stoich * Dinv[src] * alpha
    out = jax.ops.segment_sum(w2[:, None] * edge_feat[dst], src, n_nodes)
    return out + p['bias']


def kernel(gene_x, stoich, params, gpr_edge_index, met_edge_index):
    perm = jnp.argsort(gpr_edge_index[1])
    g_idx = gpr_edge_index[0][perm]
    r_idx = gpr_edge_index[1][perm]
    cnt_r = jnp.bincount(r_idx, length=N_RXN)
    H_r = _st_apply(gene_x[g_idx], r_idx, cnt_r, N_RXN, params['gr_st'])

    emb = params['emb']
    norms = jnp.linalg.norm(emb, axis=1, keepdims=True)
    Z_m = emb * jnp.where(norms > 1.0, 1.0 / (norms + 1e-7), 1.0)

    perm2 = jnp.argsort(met_edge_index[1])
    m_idx = met_edge_index[0][perm2]
    mr_idx = met_edge_index[1][perm2]
    st = stoich[perm2]
    for cp in params['convs']:
        Z_m = _conv(Z_m, m_idx, mr_idx, st, H_r, cp)

    cnt_mr = jnp.bincount(mr_idx, length=N_RXN)
    Z_r = _st_apply(Z_m[m_idx], mr_idx, cnt_mr, N_RXN, params['rm_st'])

    perm3 = jnp.argsort(gpr_edge_index[0])
    gg = gpr_edge_index[0][perm3]
    rr = gpr_edge_index[1][perm3]
    cnt_g = jnp.bincount(gg, length=N_GENES)
    Z_mg = _st_apply(Z_r[rr], gg, cnt_g, N_GENES, params['gr_st'])
    return Z_mg


# speedup vs baseline: 1.2849x; 1.1309x over previous
import jax
import jax.numpy as jnp
import numpy as np
from jax.experimental import pallas as pl
from jax.experimental.pallas import tpu as pltpu

N_GENES = 10000
N_RXN = 10000
N_MET = 10000
C = 128
NH = 2
DH = C // NH
L = 128
LS = L // 4
BS = 16

_pallas_call = pl.pallas_call


def _ln(x, w, b):
    mu = jnp.mean(x, axis=-1, keepdims=True)
    var = jnp.mean((x - mu) * (x - mu), axis=-1, keepdims=True)
    return (x - mu) * jax.lax.rsqrt(var + 1e-5) * w + b


def _mha_blk(x, y, mask_y, in_w, in_b, out_w, out_b):
    B, Lq, _ = x.shape
    Lk = y.shape[1]
    x2 = x.reshape(B * Lq, C)
    y2 = y.reshape(B * Lk, C)
    q = (x2 @ in_w[:C].T + in_b[:C]).reshape(B, Lq, C)
    k = (y2 @ in_w[C:2 * C].T + in_b[C:2 * C]).reshape(B, Lk, C)
    v = (y2 @ in_w[2 * C:].T + in_b[2 * C:]).reshape(B, Lk, C)
    acc = jnp.broadcast_to(out_b, (B * Lq, C))
    scale = 1.0 / np.sqrt(DH)
    for h in range(NH):
        qh = q[:, :, h * DH:(h + 1) * DH]
        kh = k[:, :, h * DH:(h + 1) * DH]
        vh = v[:, :, h * DH:(h + 1) * DH]
        s = jax.lax.dot_general(
            qh, kh, (((2,), (2,)), ((0,), (0,))),
            preferred_element_type=jnp.float32) * scale
        if mask_y is not None:
            s = jnp.where(mask_y[:, None, :] > 0, s, -1e9)
        s = s - jnp.max(s, axis=-1, keepdims=True)
        e = jnp.exp(s)
        a = e / jnp.sum(e, axis=-1, keepdims=True)
        oh = jax.lax.dot_general(
            a, vh, (((2,), (1,)), ((0,), (0,))),
            preferred_element_type=jnp.float32)
        acc = acc + oh.reshape(B * Lq, DH) @ out_w[:, h * DH:(h + 1) * DH].T
    return acc.reshape(B, Lq, C)


def _mab_blk(x, y, mask_x, mask_y, P, i):
    out = _mha_blk(x, y, mask_y, P['in_w'][i], P['in_b'][i],
                   P['out_w'][i], P['out_b'][i])
    if mask_x is not None:
        out = out * mask_x[:, :, None]
    out = out + x
    out = _ln(out, P['ln1_w'][i], P['ln1_b'][i])
    B, Lq, _ = out.shape
    o2 = out.reshape(B * Lq, C)
    out = out + jax.nn.relu(o2 @ P['lin_w'][i].T + P['lin_b'][i]).reshape(B, Lq, C)
    out = _ln(out, P['ln2_w'][i], P['ln2_b'][i])
    return out


def _st_chain(x, mask, P, pma_w, pma_b, seed):
    B, Lk, _ = x.shape
    for i in (0, 1):
        x = _mab_blk(x, x, mask, mask, P, i)
    xx = jax.nn.relu(x.reshape(B * Lk, C) @ pma_w[...].T + pma_b[0]).reshape(B, Lk, C)
    sd = jnp.broadcast_to(seed[...].reshape(1, 1, C), (B, 1, C))
    x = _mab_blk(sd, xx, None, mask, P, 2)
    x = _mab_blk(x, x, None, None, P, 3)
    return jnp.nan_to_num(x).reshape(B, C)


def _st_kernel(x_ref, mask_ref, in_w, in_b, out_w, out_b, lin_w, lin_b,
               l1w, l1b, l2w, l2b, pma_w, pma_b, seed, o_ref):
    x = x_ref[...]
    mask = mask_ref[...]
    P = {'in_w': in_w, 'in_b': in_b, 'out_w': out_w, 'out_b': out_b,
         'lin_w': lin_w, 'lin_b': lin_b, 'ln1_w': l1w, 'ln1_b': l1b,
         'ln2_w': l2w, 'ln2_b': l2b}
    small = jnp.max(jnp.sum(mask, axis=1)) <= LS

    @pl.when(small)
    def _():
        o_ref[...] = _st_chain(x[:, :LS, :], mask[:, :LS], P, pma_w, pma_b, seed)

    @pl.when(jnp.logical_not(small))
    def _():
        o_ref[...] = _st_chain(x, mask, P, pma_w, pma_b, seed)


def _stack_st_params(p):
    mabs = [p['encoders'][0], p['encoders'][1], p['pma_mab'], p['decoders'][0]]
    stk = lambda f: jnp.stack([m[f] for m in mabs])
    return (stk('in_w'), stk('in_b'), stk('out_w'), stk('out_b'),
            stk('lin_w'), stk('lin_b'), stk('ln1_w'), stk('ln1_b'),
            stk('ln2_w'), stk('ln2_b'),
            p['pma_lin_w'], p['pma_lin_b'].reshape(1, C),
            p['seed'].reshape(1, C))


def _st_apply(x_flat, index, counts, S, p):
    starts = jnp.cumsum(counts) - counts
    pos = jnp.arange(index.shape[0], dtype=index.dtype) - starts[index].astype(index.dtype)
    dense = jnp.zeros((S, L, C), x_flat.dtype).at[index, pos].set(x_flat)
    mask = (jnp.arange(L)[None, :] < counts[:, None]).astype(jnp.float32)
    prm = _stack_st_params(p)
    full = lambda a: pl.BlockSpec(a.shape, lambda i: (0,) * a.ndim)
    out = _pallas_call(
        _st_kernel,
        grid=(S // BS,),
        in_specs=[
            pl.BlockSpec((BS, L, C), lambda i: (i, 0, 0)),
            pl.BlockSpec((BS, L), lambda i: (i, 0)),
        ] + [full(a) for a in prm],
        out_specs=pl.BlockSpec((BS, C), lambda i: (i, 0)),
        out_shape=jax.ShapeDtypeStruct((S, C), jnp.float32),
        compiler_params=pltpu.CompilerParams(
            dimension_semantics=("parallel",)),
    )(dense, mask, *prm)
    return out


def _segment_softmax(alpha, index, num_segments):
    amax = jax.ops.segment_max(alpha, index, num_segments)
    amax = jnp.where(jnp.isfinite(amax), amax, 0.0)
    e = jnp.exp(alpha - amax[index])
    denom = jax.ops.segment_sum(e, index, num_segments) + 1e-16
    return e / denom[index]


def _conv(x, src, dst, stoich, he_attr, p):
    n_nodes = x.shape[0]
    n_edges = he_attr.shape[0]
    xt = x @ p['lin_w'].T
    he = he_attr @ p['lin_w'].T
    a_src = xt @ p['att'][:C]
    a_dst = he @ p['att'][C:]
    alpha = jax.nn.leaky_relu(a_src[src] + a_dst[dst], 0.2)
    alpha = _segment_softmax(alpha, dst, n_edges)
    ast = jnp.abs(stoich)
    Dd = jax.ops.segment_sum(ast, src, n_nodes)
    Dinv = jnp.where(Dd > 0, 1.0 / Dd, 0.0)
    Bb = jax.ops.segment_sum(ast, dst, n_edges)
    Binv = jnp.where(Bb > 0, 1.0 / Bb, 0.0)
    w1 = stoich * Binv[dst] * alpha
    edge_feat = jax.ops.segment_sum(w1[:, None] * xt[src], dst, n_edges)
    w2 = stoich * Dinv[src] * alpha
    out = jax.ops.segment_sum(w2[:, None] * edge_feat[dst], src, n_nodes)
    return out + p['bias']


def kernel(gene_x, stoich, params, gpr_edge_index, met_edge_index):
    perm = jnp.argsort(gpr_edge_index[1])
    g_idx = gpr_edge_index[0][perm]
    r_idx = gpr_edge_index[1][perm]
    cnt_r = jnp.bincount(r_idx, length=N_RXN)
    H_r = _st_apply(gene_x[g_idx], r_idx, cnt_r, N_RXN, params['gr_st'])

    emb = params['emb']
    norms = jnp.linalg.norm(emb, axis=1, keepdims=True)
    Z_m = emb * jnp.where(norms > 1.0, 1.0 / (norms + 1e-7), 1.0)

    perm2 = jnp.argsort(met_edge_index[1])
    m_idx = met_edge_index[0][perm2]
    mr_idx = met_edge_index[1][perm2]
    st = stoich[perm2]
    for cp in params['convs']:
        Z_m = _conv(Z_m, m_idx, mr_idx, st, H_r, cp)

    cnt_mr = jnp.bincount(mr_idx, length=N_RXN)
    Z_r = _st_apply(Z_m[m_idx], mr_idx, cnt_mr, N_RXN, params['rm_st'])

    perm3 = jnp.argsort(gpr_edge_index[0])
    gg = gpr_edge_index[0][perm3]
    rr = gpr_edge_index[1][perm3]
    cnt_g = jnp.bincount(gg, length=N_GENES)
    Z_mg = _st_apply(Z_r[rr], gg, cnt_g, N_GENES, params['gr_st'])
    return Z_mg


# speedup vs baseline: 1.3312x; 1.0360x over previous
import jax
import jax.numpy as jnp
import numpy as np
from jax.experimental import pallas as pl
from jax.experimental.pallas import tpu as pltpu

N_GENES = 10000
N_RXN = 10000
N_MET = 10000
C = 128
NH = 2
DH = C // NH
L = 128
LS = L // 4
BS = 40

_pallas_call = pl.pallas_call


def _ln(x, w, b):
    mu = jnp.mean(x, axis=-1, keepdims=True)
    var = jnp.mean((x - mu) * (x - mu), axis=-1, keepdims=True)
    return (x - mu) * jax.lax.rsqrt(var + 1e-5) * w + b


def _mha_blk(x, y, mask_y, in_w, in_b, out_w, out_b):
    B, Lq, _ = x.shape
    Lk = y.shape[1]
    x2 = x.reshape(B * Lq, C)
    y2 = y.reshape(B * Lk, C)
    q = (x2 @ in_w[:C].T + in_b[:C]).reshape(B, Lq, C)
    k = (y2 @ in_w[C:2 * C].T + in_b[C:2 * C]).reshape(B, Lk, C)
    v = (y2 @ in_w[2 * C:].T + in_b[2 * C:]).reshape(B, Lk, C)
    acc = jnp.broadcast_to(out_b, (B * Lq, C))
    scale = 1.0 / np.sqrt(DH)
    for h in range(NH):
        qh = q[:, :, h * DH:(h + 1) * DH]
        kh = k[:, :, h * DH:(h + 1) * DH]
        vh = v[:, :, h * DH:(h + 1) * DH]
        s = jax.lax.dot_general(
            qh, kh, (((2,), (2,)), ((0,), (0,))),
            preferred_element_type=jnp.float32) * scale
        if mask_y is not None:
            s = jnp.where(mask_y[:, None, :] > 0, s, -1e9)
        s = s - jnp.max(s, axis=-1, keepdims=True)
        e = jnp.exp(s)
        a = e / jnp.sum(e, axis=-1, keepdims=True)
        oh = jax.lax.dot_general(
            a, vh, (((2,), (1,)), ((0,), (0,))),
            preferred_element_type=jnp.float32)
        acc = acc + oh.reshape(B * Lq, DH) @ out_w[:, h * DH:(h + 1) * DH].T
    return acc.reshape(B, Lq, C)


def _mab_blk(x, y, mask_x, mask_y, P, i):
    out = _mha_blk(x, y, mask_y, P['in_w'][i], P['in_b'][i],
                   P['out_w'][i], P['out_b'][i])
    if mask_x is not None:
        out = out * mask_x[:, :, None]
    out = out + x
    out = _ln(out, P['ln1_w'][i], P['ln1_b'][i])
    B, Lq, _ = out.shape
    o2 = out.reshape(B * Lq, C)
    out = out + jax.nn.relu(o2 @ P['lin_w'][i].T + P['lin_b'][i]).reshape(B, Lq, C)
    out = _ln(out, P['ln2_w'][i], P['ln2_b'][i])
    return out


def _st_chain(x, mask, P, pma_w, pma_b, seed):
    B, Lk, _ = x.shape
    for i in (0, 1):
        x = _mab_blk(x, x, mask, mask, P, i)
    xx = jax.nn.relu(x.reshape(B * Lk, C) @ pma_w[...].T + pma_b[0]).reshape(B, Lk, C)
    sd = jnp.broadcast_to(seed[...].reshape(1, 1, C), (B, 1, C))
    x = _mab_blk(sd, xx, None, mask, P, 2)
    x = _mab_blk(x, x, None, None, P, 3)
    return jnp.nan_to_num(x).reshape(B, C)


def _st_kernel(x_ref, mask_ref, in_w, in_b, out_w, out_b, lin_w, lin_b,
               l1w, l1b, l2w, l2b, pma_w, pma_b, seed, o_ref):
    x = x_ref[...]
    mask = mask_ref[...]
    P = {'in_w': in_w, 'in_b': in_b, 'out_w': out_w, 'out_b': out_b,
         'lin_w': lin_w, 'lin_b': lin_b, 'ln1_w': l1w, 'ln1_b': l1b,
         'ln2_w': l2w, 'ln2_b': l2b}
    small = jnp.max(jnp.sum(mask, axis=1)) <= LS

    @pl.when(small)
    def _():
        o_ref[...] = _st_chain(x[:, :LS, :], mask[:, :LS], P, pma_w, pma_b, seed)

    @pl.when(jnp.logical_not(small))
    def _():
        o_ref[...] = _st_chain(x, mask, P, pma_w, pma_b, seed)


def _stack_st_params(p):
    mabs = [p['encoders'][0], p['encoders'][1], p['pma_mab'], p['decoders'][0]]
    stk = lambda f: jnp.stack([m[f] for m in mabs])
    return (stk('in_w'), stk('in_b'), stk('out_w'), stk('out_b'),
            stk('lin_w'), stk('lin_b'), stk('ln1_w'), stk('ln1_b'),
            stk('ln2_w'), stk('ln2_b'),
            p['pma_lin_w'], p['pma_lin_b'].reshape(1, C),
            p['seed'].reshape(1, C))


def _st_apply(x_flat, index, counts, S, p):
    starts = jnp.cumsum(counts) - counts
    pos = jnp.arange(index.shape[0], dtype=index.dtype) - starts[index].astype(index.dtype)
    dense = jnp.zeros((S, L, C), x_flat.dtype).at[index, pos].set(x_flat)
    mask = (jnp.arange(L)[None, :] < counts[:, None]).astype(jnp.float32)
    prm = _stack_st_params(p)
    full = lambda a: pl.BlockSpec(a.shape, lambda i: (0,) * a.ndim)
    out = _pallas_call(
        _st_kernel,
        grid=(S // BS,),
        in_specs=[
            pl.BlockSpec((BS, L, C), lambda i: (i, 0, 0)),
            pl.BlockSpec((BS, L), lambda i: (i, 0)),
        ] + [full(a) for a in prm],
        out_specs=pl.BlockSpec((BS, C), lambda i: (i, 0)),
        out_shape=jax.ShapeDtypeStruct((S, C), jnp.float32),
        compiler_params=pltpu.CompilerParams(
            dimension_semantics=("parallel",)),
    )(dense, mask, *prm)
    return out


def _segment_softmax(alpha, index, num_segments):
    amax = jax.ops.segment_max(alpha, index, num_segments)
    amax = jnp.where(jnp.isfinite(amax), amax, 0.0)
    e = jnp.exp(alpha - amax[index])
    denom = jax.ops.segment_sum(e, index, num_segments) + 1e-16
    return e / denom[index]


def _conv(x, src, dst, stoich, he_attr, p):
    n_nodes = x.shape[0]
    n_edges = he_attr.shape[0]
    xt = x @ p['lin_w'].T
    he = he_attr @ p['lin_w'].T
    a_src = xt @ p['att'][:C]
    a_dst = he @ p['att'][C:]
    alpha = jax.nn.leaky_relu(a_src[src] + a_dst[dst], 0.2)
    alpha = _segment_softmax(alpha, dst, n_edges)
    ast = jnp.abs(stoich)
    Dd = jax.ops.segment_sum(ast, src, n_nodes)
    Dinv = jnp.where(Dd > 0, 1.0 / Dd, 0.0)
    Bb = jax.ops.segment_sum(ast, dst, n_edges)
    Binv = jnp.where(Bb > 0, 1.0 / Bb, 0.0)
    w1 = stoich * Binv[dst] * alpha
    edge_feat = jax.ops.segment_sum(w1[:, None] * xt[src], dst, n_edges)
    w2 = stoich * Dinv[src] * alpha
    out = jax.ops.segment_sum(w2[:, None] * edge_feat[dst], src, n_nodes)
    return out + p['bias']


def kernel(gene_x, stoich, params, gpr_edge_index, met_edge_index):
    perm = jnp.argsort(gpr_edge_index[1])
    g_idx = gpr_edge_index[0][perm]
    r_idx = gpr_edge_index[1][perm]
    cnt_r = jnp.bincount(r_idx, length=N_RXN)
    H_r = _st_apply(gene_x[g_idx], r_idx, cnt_r, N_RXN, params['gr_st'])

    emb = params['emb']
    norms = jnp.linalg.norm(emb, axis=1, keepdims=True)
    Z_m = emb * jnp.where(norms > 1.0, 1.0 / (norms + 1e-7), 1.0)

    perm2 = jnp.argsort(met_edge_index[1])
    m_idx = met_edge_index[0][perm2]
    mr_idx = met_edge_index[1][perm2]
    st = stoich[perm2]
    for cp in params['convs']:
        Z_m = _conv(Z_m, m_idx, mr_idx, st, H_r, cp)

    cnt_mr = jnp.bincount(mr_idx, length=N_RXN)
    Z_r = _st_apply(Z_m[m_idx], mr_idx, cnt_mr, N_RXN, params['rm_st'])

    perm3 = jnp.argsort(gpr_edge_index[0])
    gg = gpr_edge_index[0][perm3]
    rr = gpr_edge_index[1][perm3]
    cnt_g = jnp.bincount(gg, length=N_GENES)
    Z_mg = _st_apply(Z_r[rr], gg, cnt_g, N_GENES, params['gr_st'])
    return Z_mg
